# baseline (speedup 1.0000x reference)
import functools

import jax
import jax.numpy as jnp
from jax.experimental import pallas as pl
from jax.experimental.pallas import tpu as pltpu

N = 1000000
EPS = 1e-07

_BOX_ROWS = 4000
_BOX_COLS = 1000
_SC_COLS = 250
_BLK_ROWS = 160
_GRID = _BOX_ROWS // _BLK_ROWS


def _loss_kernel(obb_ref, tbb_ref, os_ref, ts_ref, out_ref, acc_ref, e_ref):
    step = pl.program_id(0)

    @pl.when(step == 0)
    def _init():
        for i in range(5):
            acc_ref[i] = 0.0
        row = jax.lax.broadcasted_iota(jnp.int32, (_BOX_COLS, _SC_COLS), 0)
        col = jax.lax.broadcasted_iota(jnp.int32, (_BOX_COLS, _SC_COLS), 1)
        e_ref[...] = jnp.where(
            jax.lax.shift_right_logical(row, 2) == col, 1.0, 0.0)

    t = ts_ref[...]
    o = jnp.clip(os_ref[...], EPS, 1.0 - EPS)

    sel = jnp.where(t == 1.0, o, jnp.where(t == 0.0, 1.0 - o, 1.0))
    q = _BLK_ROWS // 4
    sel4 = sel[0:q] * sel[q:2 * q] * sel[2 * q:3 * q] * sel[3 * q:4 * q]
    log_part = jnp.sum(jnp.log(sel4))
    cnt_part = jnp.sum(jnp.where(t != -1.0, 1.0, 0.0))

    raw = os_ref[...]
    mask_r = jnp.where(raw != -1.0, 1.0, 0.0)
    w = jnp.where(raw > 0.0, mask_r, 0.0)
    b_part = jnp.sum(w)
    k_part = jnp.sum(mask_r)

    d = jnp.abs(obb_ref[...] - tbb_ref[...])
    sl = jnp.where(d < 1.0, 0.5 * d * d, d - 0.5)
    ay = jax.lax.dot_general(
        sl, e_ref[...],
        dimension_numbers=(((1,), (0,)), ((), ())),
        preferred_element_type=jnp.float32)
    a_part = jnp.sum(w * ay)

    acc_ref[0] += log_part
    acc_ref[1] += cnt_part
    acc_ref[2] += a_part
    acc_ref[3] += b_part
    acc_ref[4] += k_part

    @pl.when(step == _GRID - 1)
    def _finish():
        cls = (-acc_ref[0]) / jnp.maximum(acc_ref[1], 1.0)
        reg = acc_ref[2] / (acc_ref[3] + EPS * acc_ref[4])
        out_ref[0] = cls + reg


@functools.partial(jax.jit, static_argnums=())
def _loss(obb, tbb, os_, ts):
    obb_f = obb.reshape(_BOX_ROWS, _BOX_COLS)
    tbb_f = tbb.reshape(_BOX_ROWS, _BOX_COLS)
    os_f = os_.reshape(_BOX_ROWS, _SC_COLS)
    ts_f = ts.reshape(_BOX_ROWS, _SC_COLS)
    out = pl.pallas_call(
        _loss_kernel,
        grid=(_GRID,),
        in_specs=[
            pl.BlockSpec((_BLK_ROWS, _BOX_COLS), lambda i: (i, 0)),
            pl.BlockSpec((_BLK_ROWS, _BOX_COLS), lambda i: (i, 0)),
            pl.BlockSpec((_BLK_ROWS, _SC_COLS), lambda i: (i, 0)),
            pl.BlockSpec((_BLK_ROWS, _SC_COLS), lambda i: (i, 0)),
        ],
        out_specs=pl.BlockSpec(memory_space=pltpu.SMEM),
        out_shape=jax.ShapeDtypeStruct((1,), jnp.float32),
        scratch_shapes=[pltpu.SMEM((8,), jnp.float32),
                        pltpu.VMEM((_BOX_COLS, _SC_COLS), jnp.float32)],
    )(obb_f, tbb_f, os_f, ts_f)
    return out[0]


def kernel(output_bounding_boxes, target_bounding_boxes, output_scores, target_scores):
    loss = _loss(output_bounding_boxes, target_bounding_boxes,
                 output_scores, target_scores)
    obb = output_bounding_boxes.reshape(1, -1, 4)
    os_ = output_scores.reshape(1, -1)
    return (obb, os_, loss)


# speedup vs baseline: 10.1530x; 10.1530x over previous
import jax
import jax.numpy as jnp
from jax import lax
from jax.experimental import pallas as pl
from jax.experimental.pallas import tpu as pltpu
from jax.experimental.pallas import tpu_sc as plsc

EPS = 1e-07
N = 1000000
NW = 32
CHUNK = 4000
W_CHUNKS = 8
GROUPS = CHUNK // 32

_LN2 = 0.6931471805599453
_SQRT2 = 1.4142135623730951
_LOG_P = (7.0376836292e-2, -1.1514610310e-1, 1.1676998740e-1,
          -1.2420140846e-1, 1.4249322787e-1, -1.6668057665e-1,
          2.0000714765e-1, -2.4999993993e-1, 3.3333331174e-1)


def _log16(x):
    i = plsc.bitcast(x, jnp.int32)
    e = (lax.shift_right_logical(i, 23) - 127).astype(jnp.float32)
    m = plsc.bitcast((i & 0x007FFFFF) | 0x3F800000, jnp.float32)
    big = m > _SQRT2
    m = jnp.where(big, 0.5 * m, m)
    e = jnp.where(big, e + 1.0, e)
    u = m - 1.0
    p = jnp.full_like(u, _LOG_P[0])
    for c in _LOG_P[1:]:
        p = p * u + c
    z = u * u
    return u + (u * z * p - 0.5 * z) + e * _LN2


def _sc_body(obb_hbm, tbb_hbm, os_hbm, ts_hbm, out_hbm,
             obb_v, tbb_v, os_v, ts_v, res_v, sem):
    wid = lax.axis_index("s") * 2 + lax.axis_index("c")
    nc = jnp.where(wid == NW - 1, 2, W_CHUNKS)

    def copies(slot, par):
        r = wid * W_CHUNKS + slot
        cs = []
        for j in range(4):
            cs.append((obb_hbm.at[j * (N // CHUNK) + r],
                       obb_v.at[pl.ds((par * 4 + j) * CHUNK, CHUNK)],
                       sem.at[par, j]))
            cs.append((tbb_hbm.at[j * (N // CHUNK) + r],
                       tbb_v.at[pl.ds((par * 4 + j) * CHUNK, CHUNK)],
                       sem.at[par, 4 + j]))
        cs.append((os_hbm.at[r], os_v.at[pl.ds(par * CHUNK, CHUNK)],
                   sem.at[par, 8]))
        cs.append((ts_hbm.at[r], ts_v.at[pl.ds(par * CHUNK, CHUNK)],
                   sem.at[par, 9]))
        return cs

    def issue(slot, par):
        for src, dst, s in copies(slot, par):
            pltpu.async_copy(src, dst, s)

    def wait_all(slot, par):
        for src, dst, s in copies(slot, par):
            pltpu.make_async_copy(src, dst, s).wait()

    zero = jnp.zeros((16,), jnp.float32)
    for k in range(4):
        res_v[pl.ds(16 * k, 16)] = zero

    issue(0, 0)
    for slot in range(W_CHUNKS):
        par = slot % 2
        if slot + 1 < W_CHUNKS:

            @pl.when(slot + 1 < nc)
            def _issue_next(slot=slot):
                issue(slot + 1, (slot + 1) % 2)

        @pl.when(slot < nc)
        def _compute(slot=slot, par=par):
            wait_all(slot, par)
            so = par * CHUNK
            bo = par * 4 * CHUNK

            def group(g, acc):
                log_s, cnt_s, a_s, b_s = acc
                sel_prod = None
                for h2 in range(2):
                    s = so + g * 32 + h2 * 16
                    sb = bo + g * 32 + h2 * 16
                    o_raw = os_v[pl.ds(s, 16)]
                    t = ts_v[pl.ds(s, 16)]
                    o = jnp.clip(o_raw, EPS, 1.0 - EPS)
                    sel = jnp.where(t == 1.0, o,
                                    jnp.where(t == 0.0, 1.0 - o, 1.0))
                    sel_prod = sel if sel_prod is None else sel_prod * sel
                    cnt_s = cnt_s + jnp.minimum(t + 1.0, 1.0)
                    w = jnp.where(o_raw > 0.0, 1.0, 0.0)
                    ay = None
                    for j in range(4):
                        d = jnp.abs(obb_v[pl.ds(sb + j * CHUNK, 16)]
                                    - tbb_v[pl.ds(sb + j * CHUNK, 16)])
                        sl = jnp.where(d < 1.0, 0.5 * d * d, d - 0.5)
                        ay = sl if ay is None else ay + sl
                    a_s = a_s + w * ay
                    b_s = b_s + w
                log_s = log_s + _log16(sel_prod)
                return (log_s, cnt_s, a_s, b_s)

            acc0 = tuple(res_v[pl.ds(16 * k, 16)] for k in range(4))
            log_s, cnt_s, a_s, b_s = lax.fori_loop(0, GROUPS, group, acc0)
            res_v[pl.ds(0, 16)] = log_s
            res_v[pl.ds(16, 16)] = cnt_s
            res_v[pl.ds(32, 16)] = a_s
            res_v[pl.ds(48, 16)] = b_s

    pltpu.sync_copy(res_v, out_hbm.at[wid])


def _sc_partials(obb_cm, tbb_cm, os_flat, ts_flat):
    mesh = plsc.VectorSubcoreMesh(core_axis_name="c", subcore_axis_name="s")
    return pl.kernel(
        _sc_body,
        out_type=jax.ShapeDtypeStruct((NW, 64), jnp.float32),
        mesh=mesh,
        scratch_types=[
            pltpu.VMEM((8 * CHUNK,), jnp.float32),
            pltpu.VMEM((8 * CHUNK,), jnp.float32),
            pltpu.VMEM((2 * CHUNK,), jnp.float32),
            pltpu.VMEM((2 * CHUNK,), jnp.float32),
            pltpu.VMEM((64,), jnp.float32),
            pltpu.SemaphoreType.DMA((2, 10)),
        ],
        compiler_params=pltpu.CompilerParams(needs_layout_passes=False,
                                             use_tc_tiling_on_sc=False),
    )(obb_cm, tbb_cm, os_flat, ts_flat)


def kernel(output_bounding_boxes, target_bounding_boxes, output_scores, target_scores):
    obb_cm = jnp.transpose(output_bounding_boxes, (0, 2, 1)).reshape(4 * N // CHUNK, CHUNK)
    tbb_cm = jnp.transpose(target_bounding_boxes, (0, 2, 1)).reshape(4 * N // CHUNK, CHUNK)
    os_flat = output_scores.reshape(N // CHUNK, CHUNK)
    ts_flat = target_scores.reshape(N // CHUNK, CHUNK)
    parts = _sc_partials(obb_cm, tbb_cm, os_flat, ts_flat)
    p = parts.reshape(NW, 4, 16).sum(axis=(0, 2))
    cls = (-p[0]) / jnp.maximum(p[1], 1.0)
    reg = p[2] / (p[3] + EPS * float(N))
    loss = cls + reg
    obb = output_bounding_boxes.reshape(1, -1, 4)
    os_ = output_scores.reshape(1, -1)
    return (obb, os_, loss)
